# baseline (speedup 1.0000x reference)
import functools

import jax
import jax.numpy as jnp
from jax.experimental import pallas as pl
from jax.experimental.pallas import tpu as pltpu


def _fused_kernel(h_ref, p_ref, a_ref,
                  w_out, act_out, ent_out, idx_out,
                  sim_scr, hn_scr, n_scr, cols_scr,
                  act_scr, *, kt, nk):
    i = pl.program_id(0)
    ph = pl.program_id(1)
    k = pl.program_id(2)

    @pl.when((ph == 0) & (k == 0))
    def _init():
        h = h_ref[...]
        n = jnp.sqrt(jnp.sum(h * h, axis=1, keepdims=True))
        hn_scr[...] = h / jnp.clip(n, 1e-12, None)
        lane = jax.lax.broadcasted_iota(jnp.int32, cols_scr.shape, 1)
        cols_scr[...] = jnp.where(lane == 2, -jnp.inf, 0.0)

    @pl.when(ph == 0)
    def _phase0():
        p = p_ref[...]

        @pl.when(i == 0)
        def _():
            n_scr[k] = jnp.sqrt(jnp.sum(p * p, axis=1, keepdims=True))

        p = p / jnp.clip(n_scr[k], 1e-12, None)
        sim = jax.lax.dot_general(
            hn_scr[...], p,
            dimension_numbers=(((1,), (1,)), ((), ())),
            preferred_element_type=jnp.float32,
        )
        e = jnp.exp(sim)
        sim_scr[k] = e
        cols_scr[:, 0:1] += jnp.sum(e, axis=1, keepdims=True)
        cols_scr[:, 1:2] += jnp.sum(e * sim, axis=1, keepdims=True)

    @pl.when(ph == 1)
    def _phase1():
        inv_s = 1.0 / cols_scr[:, 0:1]
        w = sim_scr[k] * inv_s
        w_out[...] = w
        tmax = jnp.max(w, axis=1, keepdims=True)
        iota = jax.lax.broadcasted_iota(jnp.int32, w.shape, 1)
        targ = jnp.min(jnp.where(w == tmax, iota, jnp.int32(kt)),
                       axis=1, keepdims=True)
        better = tmax > cols_scr[:, 2:3]
        cols_scr[:, 3:4] = jnp.where(
            better, (k * kt + targ).astype(jnp.float32), cols_scr[:, 3:4])
        cols_scr[:, 2:3] = jnp.maximum(cols_scr[:, 2:3], tmax)
        part = jax.lax.dot_general(
            w, a_ref[...],
            dimension_numbers=(((1,), (0,)), ((), ())),
            preferred_element_type=jnp.float32,
        )

        @pl.when(k == 0)
        def _():
            act_scr[...] = part

        @pl.when(k > 0)
        def _():
            act_scr[...] += part

        @pl.when(k == nk - 1)
        def _fin():
            act_out[...] = act_scr[...]
            ent_out[...] = (jnp.log(cols_scr[:, 0:1])
                            - cols_scr[:, 1:2] * inv_s)
            idx_out[...] = cols_scr[:, 3:4].astype(jnp.int32)


def kernel(intent_emb, l1_protos, l1_actions):
    b, d = intent_emb.shape
    kl1, _ = l1_protos.shape
    adim = l1_actions.shape[1]

    bt = min(1024, b)
    kt = min(512, kl1)
    nb = b // bt
    nk = kl1 // kt

    grid = (nb, 2, nk)

    w, action, ent_rows, idx = pl.pallas_call(
        functools.partial(_fused_kernel, kt=kt, nk=nk),
        grid=grid,
        in_specs=[
            pl.BlockSpec((bt, d), lambda i, p, k: (i, 0)),
            pl.BlockSpec((kt, d), lambda i, p, k: (jnp.where(p == 0, k, nk - 1), 0)),
            pl.BlockSpec((kt, adim), lambda i, p, k: (jnp.where(p == 1, k, 0), 0)),
        ],
        out_specs=[
            pl.BlockSpec((bt, kt), lambda i, p, k: (i, jnp.where(p == 1, k, 0))),
            pl.BlockSpec((bt, adim), lambda i, p, k: (i, 0)),
            pl.BlockSpec((bt, 1), lambda i, p, k: (i, 0)),
            pl.BlockSpec((bt, 1), lambda i, p, k: (i, 0)),
        ],
        out_shape=[
            jax.ShapeDtypeStruct((b, kl1), jnp.float32),
            jax.ShapeDtypeStruct((b, adim), jnp.float32),
            jax.ShapeDtypeStruct((b, 1), jnp.float32),
            jax.ShapeDtypeStruct((b, 1), jnp.int32),
        ],
        scratch_shapes=[
            pltpu.VMEM((nk, bt, kt), jnp.float32),
            pltpu.VMEM((bt, d), jnp.float32),
            pltpu.VMEM((nk, kt, 1), jnp.float32),
            pltpu.VMEM((bt, 128), jnp.float32),
            pltpu.VMEM((bt, adim), jnp.float32),
        ],
        compiler_params=pltpu.CompilerParams(
            dimension_semantics=("arbitrary", "arbitrary", "arbitrary"),
            vmem_limit_bytes=110 * 1024 * 1024,
        ),
    )(intent_emb, l1_protos, l1_actions)

    ent = jnp.mean(ent_rows)
    return (action, w, ent, idx.reshape(b))


# speedup vs baseline: 1.0876x; 1.0876x over previous
import functools

import jax
import jax.numpy as jnp
from jax.experimental import pallas as pl
from jax.experimental.pallas import tpu as pltpu


def _fused_kernel(h_ref, p_ref, a_ref,
                  w_out, act_out, ent_out, idx_out,
                  e_scr, hn_scr, s_scr, dot_scr, m_scr, amax_scr, act_scr,
                  *, kt, nk):
    ph = pl.program_id(1)
    k = pl.program_id(2)

    @pl.when((ph == 0) & (k == 0))
    def _init():
        h = h_ref[...]
        n = jnp.sqrt(jnp.sum(h * h, axis=1, keepdims=True))
        hn_scr[...] = h / jnp.clip(n, 1e-12, None)
        s_scr[...] = jnp.zeros_like(s_scr)
        dot_scr[...] = jnp.zeros_like(dot_scr)
        m_scr[...] = jnp.full_like(m_scr, -jnp.inf)
        amax_scr[...] = jnp.zeros_like(amax_scr)

    @pl.when(ph == 0)
    def _phase0():
        p = p_ref[...]
        pn = jnp.sqrt(jnp.sum(p * p, axis=1, keepdims=True))
        p = p / jnp.clip(pn, 1e-12, None)
        sim = jax.lax.dot_general(
            hn_scr[...], p,
            dimension_numbers=(((1,), (1,)), ((), ())),
            preferred_element_type=jnp.float32,
        )
        e = jnp.exp(sim)
        e_scr[k] = e
        s_scr[...] += jnp.sum(e, axis=1, keepdims=True)
        dot_scr[...] += jnp.sum(e * sim, axis=1, keepdims=True)

    @pl.when(ph == 1)
    def _phase1():
        inv_s = 1.0 / s_scr[...]
        w = e_scr[k] * inv_s
        w_out[...] = w
        tmax = jnp.max(w, axis=1, keepdims=True)
        iota = jax.lax.broadcasted_iota(jnp.int32, w.shape, 1)
        targ = jnp.min(jnp.where(w == tmax, iota, jnp.int32(kt)),
                       axis=1, keepdims=True)
        better = tmax > m_scr[...]
        amax_scr[...] = jnp.where(better, k * kt + targ, amax_scr[...])
        m_scr[...] = jnp.maximum(m_scr[...], tmax)
        part = jax.lax.dot_general(
            w, a_ref[...],
            dimension_numbers=(((1,), (0,)), ((), ())),
            preferred_element_type=jnp.float32,
        )

        @pl.when(k == 0)
        def _():
            act_scr[...] = part

        @pl.when(k > 0)
        def _():
            act_scr[...] += part

        @pl.when(k == nk - 1)
        def _fin():
            act_out[...] = act_scr[...]
            ent_out[...] = jnp.log(s_scr[...]) - dot_scr[...] * inv_s
            idx_out[...] = amax_scr[...]


def kernel(intent_emb, l1_protos, l1_actions):
    b, d = intent_emb.shape
    kl1, _ = l1_protos.shape
    adim = l1_actions.shape[1]

    bt = min(1024, b)
    kt = min(512, kl1)
    nb = b // bt
    nk = kl1 // kt

    grid = (nb, 2, nk)

    w, action, ent_rows, idx = pl.pallas_call(
        functools.partial(_fused_kernel, kt=kt, nk=nk),
        grid=grid,
        in_specs=[
            pl.BlockSpec((bt, d), lambda i, p, k: (i, 0)),
            pl.BlockSpec((kt, d), lambda i, p, k: (jnp.where(p == 0, k, nk - 1), 0)),
            pl.BlockSpec((kt, adim), lambda i, p, k: (jnp.where(p == 1, k, 0), 0)),
        ],
        out_specs=[
            pl.BlockSpec((bt, kt), lambda i, p, k: (i, jnp.where(p == 1, k, 0))),
            pl.BlockSpec((bt, adim), lambda i, p, k: (i, 0)),
            pl.BlockSpec((bt, 1), lambda i, p, k: (i, 0)),
            pl.BlockSpec((bt, 1), lambda i, p, k: (i, 0)),
        ],
        out_shape=[
            jax.ShapeDtypeStruct((b, kl1), jnp.float32),
            jax.ShapeDtypeStruct((b, adim), jnp.float32),
            jax.ShapeDtypeStruct((b, 1), jnp.float32),
            jax.ShapeDtypeStruct((b, 1), jnp.int32),
        ],
        scratch_shapes=[
            pltpu.VMEM((nk, bt, kt), jnp.float32),
            pltpu.VMEM((bt, d), jnp.float32),
            pltpu.VMEM((bt, 1), jnp.float32),
            pltpu.VMEM((bt, 1), jnp.float32),
            pltpu.VMEM((bt, 1), jnp.float32),
            pltpu.VMEM((bt, 1), jnp.int32),
            pltpu.VMEM((bt, adim), jnp.float32),
        ],
        compiler_params=pltpu.CompilerParams(
            dimension_semantics=("parallel", "arbitrary", "arbitrary"),
            vmem_limit_bytes=110 * 1024 * 1024,
        ),
    )(intent_emb, l1_protos, l1_actions)

    ent = jnp.mean(ent_rows)
    return (action, w, ent, idx.reshape(b))


# speedup vs baseline: 1.2778x; 1.1749x over previous
import functools

import jax
import jax.numpy as jnp
from jax.experimental import pallas as pl
from jax.experimental.pallas import tpu as pltpu


def _fused_kernel(h_ref, p_ref, a_ref,
                  w_out, act_out, ent_out, idx_out,
                  e_scr, hn_scr, s_scr, dot_scr, m_scr, amax_scr, act_scr,
                  copy_sem, *, bt, kt, nk):
    i = pl.program_id(0)
    ph = pl.program_id(1)
    k = pl.program_id(2)

    @pl.when((ph == 0) & (k == 0))
    def _init():
        cp = pltpu.make_async_copy(
            h_ref.at[pl.ds(i * bt, bt), :], hn_scr, copy_sem)
        cp.start()
        cp.wait()
        h = hn_scr[...]
        n = jnp.sqrt(jnp.sum(h * h, axis=1, keepdims=True))
        hn_scr[...] = h / jnp.clip(n, 1e-12, None)
        s_scr[...] = jnp.zeros_like(s_scr)
        dot_scr[...] = jnp.zeros_like(dot_scr)
        m_scr[...] = jnp.full_like(m_scr, -jnp.inf)
        amax_scr[...] = jnp.zeros_like(amax_scr)

    @pl.when(ph == 0)
    def _phase0():
        p = p_ref[...]
        pn = jnp.sqrt(jnp.sum(p * p, axis=1, keepdims=True))
        p = p / jnp.clip(pn, 1e-12, None)
        sim = jax.lax.dot_general(
            hn_scr[...], p,
            dimension_numbers=(((1,), (1,)), ((), ())),
            preferred_element_type=jnp.float32,
        )
        e = jnp.exp(sim)
        e_scr[k] = e
        s_scr[...] += jnp.sum(e, axis=1, keepdims=True)
        dot_scr[...] += jnp.sum(e * sim, axis=1, keepdims=True)

    @pl.when(ph == 1)
    def _phase1():
        inv_s = 1.0 / s_scr[...]
        w = e_scr[k] * inv_s
        w_out[...] = w
        tmax = jnp.max(w, axis=1, keepdims=True)
        iota = jax.lax.broadcasted_iota(jnp.int32, w.shape, 1)
        targ = jnp.min(jnp.where(w == tmax, iota, jnp.int32(kt)),
                       axis=1, keepdims=True)
        better = tmax > m_scr[...]
        amax_scr[...] = jnp.where(better, k * kt + targ, amax_scr[...])
        m_scr[...] = jnp.maximum(m_scr[...], tmax)
        part = jax.lax.dot_general(
            w, a_ref[...],
            dimension_numbers=(((1,), (0,)), ((), ())),
            preferred_element_type=jnp.float32,
        )

        @pl.when(k == 0)
        def _():
            act_scr[...] = part

        @pl.when(k > 0)
        def _():
            act_scr[...] += part

        @pl.when(k == nk - 1)
        def _fin():
            act_out[...] = act_scr[...]
            ent_out[...] = jnp.log(s_scr[...]) - dot_scr[...] * inv_s
            idx_out[...] = amax_scr[...]


def kernel(intent_emb, l1_protos, l1_actions):
    b, d = intent_emb.shape
    kl1, _ = l1_protos.shape
    adim = l1_actions.shape[1]

    bt = min(1024, b)
    kt = min(1024, kl1)
    nb = b // bt
    nk = kl1 // kt

    grid = (nb, 2, nk)

    w, action, ent_rows, idx = pl.pallas_call(
        functools.partial(_fused_kernel, bt=bt, kt=kt, nk=nk),
        grid=grid,
        in_specs=[
            pl.BlockSpec(memory_space=pltpu.MemorySpace.HBM),
            pl.BlockSpec((kt, d), lambda i, p, k: (jnp.where(p == 0, k, nk - 1), 0)),
            pl.BlockSpec((kt, adim), lambda i, p, k: (jnp.where(p == 1, k, 0), 0)),
        ],
        out_specs=[
            pl.BlockSpec((bt, kt), lambda i, p, k: (i, jnp.where(p == 1, k, 0))),
            pl.BlockSpec((bt, adim), lambda i, p, k: (i, 0)),
            pl.BlockSpec((bt, 1), lambda i, p, k: (i, 0)),
            pl.BlockSpec((bt, 1), lambda i, p, k: (i, 0)),
        ],
        out_shape=[
            jax.ShapeDtypeStruct((b, kl1), jnp.float32),
            jax.ShapeDtypeStruct((b, adim), jnp.float32),
            jax.ShapeDtypeStruct((b, 1), jnp.float32),
            jax.ShapeDtypeStruct((b, 1), jnp.int32),
        ],
        scratch_shapes=[
            pltpu.VMEM((nk, bt, kt), jnp.float32),
            pltpu.VMEM((bt, d), jnp.float32),
            pltpu.VMEM((bt, 1), jnp.float32),
            pltpu.VMEM((bt, 1), jnp.float32),
            pltpu.VMEM((bt, 1), jnp.float32),
            pltpu.VMEM((bt, 1), jnp.int32),
            pltpu.VMEM((bt, adim), jnp.float32),
            pltpu.SemaphoreType.DMA,
        ],
        compiler_params=pltpu.CompilerParams(
            dimension_semantics=("parallel", "arbitrary", "arbitrary"),
            vmem_limit_bytes=110 * 1024 * 1024,
        ),
    )(intent_emb, l1_protos, l1_actions)

    ent = jnp.mean(ent_rows)
    return (action, w, ent, idx.reshape(b))
